# baseline (speedup 1.0000x reference)
import functools
import math

import jax
import jax.numpy as jnp
from jax import lax
from jax.experimental import pallas as pl
from jax.experimental.pallas import tpu as pltpu
from jax.experimental.pallas import tpu_sc as plsc

_D = 1024
_DH = _D // 2
_SEQ = 4096
_THETA = 10000.0
_SCALE = math.sqrt(float(_D))

_KP = 8
_NBUF = 3

_MAGIC = 12582912.0
_TWO_OVER_PI = 0.6366197723675814
_PIO2_HI = 1.57079637050628662109375
_PIO2_LO = -4.37113900018624283e-8
_C = [2.443315711809948e-5 * _SCALE, -1.388731625493765e-3 * _SCALE,
      4.166664568298827e-2 * _SCALE, -0.5 * _SCALE, _SCALE]
_S = [-1.9515295891e-4 * _SCALE, 8.3321608736e-3 * _SCALE,
      -1.6666654611e-1 * _SCALE, _SCALE]


@functools.cache
def _make_sc_kernel(batch):
    info = plsc.get_sparse_core_info()
    nc, ns, L = info.num_cores, info.num_subcores, info.num_lanes
    nw = nc * ns
    ppw = _SEQ // nw
    steps = ppw // _KP
    kt = batch * _KP
    B = batch * _SEQ

    mesh = plsc.VectorSubcoreMesh(core_axis_name="c", subcore_axis_name="s")

    @functools.partial(
        pl.kernel,
        mesh=mesh,
        out_type=jax.ShapeDtypeStruct((B, _D), jnp.float32),
        scratch_types=(
            [pltpu.VMEM((steps, kt), jnp.int32),
             pltpu.VMEM((_DH,), jnp.float32)]
            + [pltpu.VMEM((kt, _D), jnp.float32)] * _NBUF
            + [pltpu.SemaphoreType.DMA] * (2 * _NBUF)
        ),
    )
    def sc(ids_hbm, table_hbm, out_hbm, idx_v, freq_v, *bufs):
        rows = bufs[0:_NBUF]
        sem_g = bufs[_NBUF:2 * _NBUF]
        sem_s = bufs[2 * _NBUF:3 * _NBUF]

        wid = lax.axis_index("s") * nc + lax.axis_index("c")
        pos_base = wid * ppw

        pltpu.sync_copy(ids_hbm.at[wid], idx_v)

        def fire(s, q):
            pltpu.async_copy(table_hbm.at[idx_v.at[s]], rows[q], sem_g[q])

        def wait_in(q):
            pltpu.make_async_copy(
                table_hbm.at[idx_v.at[0]], rows[q], sem_g[q]).wait()

        def fire_store(s, q):
            for b in range(batch):
                pltpu.async_copy(
                    rows[q].at[pl.ds(b * _KP, _KP)],
                    out_hbm.at[pl.ds(b * _SEQ + pos_base + s * _KP, _KP), :],
                    sem_s[q])

        def wait_store(q):
            for b in range(batch):
                pltpu.make_async_copy(
                    rows[q].at[pl.ds(b * _KP, _KP)],
                    out_hbm.at[pl.ds(b * _SEQ, _KP), :],
                    sem_s[q]).wait()

        def init_freq(h, carry):
            j = (lax.iota(jnp.int32, L) + h * L).astype(jnp.float32)
            freq_v[pl.ds(h * L, L)] = jnp.exp(
                j * (-2.0 * math.log(_THETA) / _D))
            return carry

        lax.fori_loop(0, _DH // L, init_freq, 0)

        def compute(s, q):
            rq = rows[q]

            def body(h, carry):
                o = h * L
                om = freq_v[pl.ds(o, L)]
                for j in range(_KP):
                    p = (pos_base + s * _KP + j).astype(jnp.float32)
                    ang = om * p
                    kf = (ang * _TWO_OVER_PI + _MAGIC) - _MAGIC
                    ki = kf.astype(jnp.int32)
                    r = (ang - kf * _PIO2_HI) - kf * _PIO2_LO
                    z = r * r
                    cv = (((_C[0] * z + _C[1]) * z + _C[2]) * z + _C[3]) \
                        * z + _C[4]
                    sv = r * (((_S[0] * z + _S[1]) * z + _S[2]) * z + _S[3])
                    b0 = (ki & 1) != 0
                    b1 = (ki & 2) != 0
                    cq = jnp.where(b0, -sv, cv)
                    sq = jnp.where(b0, cv, sv)
                    cq = jnp.where(b1, -cq, cq)
                    sq = jnp.where(b1, -sq, sq)
                    for b in range(batch):
                        t = b * _KP + j
                        x1 = rq[t, pl.ds(o, L)]
                        x2 = rq[t, pl.ds(_DH + o, L)]
                        rq[t, pl.ds(o, L)] = x1 * cq - x2 * sq
                        rq[t, pl.ds(_DH + o, L)] = x2 * cq + x1 * sq
                return carry

            lax.fori_loop(0, _DH // L, body, 0)

        for s in range(_NBUF - 1):
            fire(s, s % _NBUF)
        for s in range(steps):
            q = s % _NBUF
            wait_in(q)
            ns = s + _NBUF - 1
            if ns < steps:
                if ns >= _NBUF:
                    wait_store(ns % _NBUF)
                fire(ns, ns % _NBUF)
            compute(s, q)
            fire_store(s, q)
        for s in range(steps - _NBUF, steps):
            wait_store(s % _NBUF)

    return sc


def kernel(token_ids, table):
    batch, seq = token_ids.shape
    nw = 32
    ids = token_ids.astype(jnp.int32).reshape(
        batch, nw, seq // nw // _KP, _KP).transpose(1, 2, 0, 3).reshape(
        nw, seq // nw // _KP, batch * _KP)
    out = _make_sc_kernel(batch)(ids, table)
    return out.reshape(batch, seq, _D)
